# Initial kernel scaffold; baseline (speedup 1.0000x reference)
import jax
import jax.numpy as jnp
from jax.experimental import pallas as pl


def kernel(ref_points, points):
    raise NotImplementedError("write your pallas kernel here")


# speedup vs baseline: 179.2171x; 179.2171x over previous
import jax
import jax.numpy as jnp
from jax.experimental import pallas as pl
from jax.experimental.pallas import tpu as pltpu

_NN = 16
_BALL2 = 0.2
_B, _N, _C = 4, 4096, 3
_RB = 256


def _beta_max(r):
    beta = jnp.full_like(r, 2.0)
    for _ in range(24):
        f = beta * beta * beta - 3.0 * beta - 2.0 * r
        fp = 3.0 * beta * beta - 3.0
        beta = jnp.clip(beta - f / (fp + 1e-12), 1.0, 2.0)
    return beta


def _cond_from_gram(axx, ayy, azz, axy, axz, ayz):
    q = (axx + ayy + azz) * (1.0 / 3.0)
    p1 = axy * axy + axz * axz + ayz * ayz
    dxx = axx - q
    dyy = ayy - q
    dzz = azz - q
    p2 = dxx * dxx + dyy * dyy + dzz * dzz + 2.0 * p1
    p = jnp.sqrt(jnp.maximum(p2 * (1.0 / 6.0), 0.0))
    pinv = jnp.where(p > 1e-30, 1.0 / jnp.maximum(p, 1e-30), 0.0)
    bxx = dxx * pinv
    byy = dyy * pinv
    bzz = dzz * pinv
    bxy = axy * pinv
    bxz = axz * pinv
    byz = ayz * pinv
    detb = (bxx * (byy * bzz - byz * byz)
            - bxy * (bxy * bzz - byz * bxz)
            + bxz * (bxy * byz - byy * bxz))
    r = jnp.clip(0.5 * detb, -1.0, 1.0)
    lmax = q + p * _beta_max(r)
    lmin = q - p * _beta_max(-r)
    s0 = jnp.sqrt(jnp.maximum(lmax, 0.0))
    s2 = jnp.sqrt(jnp.maximum(lmin, 0.0))
    return s0 / (s0 + s2 + 1e-30)


def _body(refc_ref, reft_ref, ptsc_ref, out_ref):
    b = pl.program_id(0)
    i = pl.program_id(1)

    tx = refc_ref[:, 0:1]
    ty = refc_ref[:, 1:2]
    tz = refc_ref[:, 2:3]
    ax = reft_ref[0:1, :]
    ay = reft_ref[1:2, :]
    az = reft_ref[2:3, :]

    dx = tx - ax
    dy = ty - ay
    dz = tz - az
    d = dx * dx + dy * dy + dz * dz

    dwork = d
    t = None
    for it in range(_NN):
        t = jnp.min(dwork, axis=0, keepdims=True)
        if it < _NN - 1:
            dwork = jnp.where(dwork == t, jnp.inf, dwork)

    sel = (d <= t).astype(jnp.float32)
    m = jnp.where(d < _BALL2, sel, 0.0)
    nb = jnp.sum(m, axis=0, keepdims=True)

    m1x = jnp.sum(m * dx, axis=0, keepdims=True)
    m1y = jnp.sum(m * dy, axis=0, keepdims=True)
    m1z = jnp.sum(m * dz, axis=0, keepdims=True)
    m2xx = jnp.sum(m * dx * dx, axis=0, keepdims=True)
    m2yy = jnp.sum(m * dy * dy, axis=0, keepdims=True)
    m2zz = jnp.sum(m * dz * dz, axis=0, keepdims=True)
    m2xy = jnp.sum(m * dx * dy, axis=0, keepdims=True)
    m2xz = jnp.sum(m * dx * dz, axis=0, keepdims=True)
    m2yz = jnp.sum(m * dy * dz, axis=0, keepdims=True)
    nbinv = 1.0 / nb
    csx = m1x * nbinv
    csy = m1y * nbinv
    csz = m1z * nbinv
    cx = csx + ax
    cy = csy + ay
    cz = csz + az
    w = _NN - nb
    gxx = m2xx - nb * csx * csx + w * cx * cx
    gyy = m2yy - nb * csy * csy + w * cy * cy
    gzz = m2zz - nb * csz * csz + w * cz * cz
    gxy = m2xy - nb * csx * csy + w * cx * cy
    gxz = m2xz - nb * csx * csz + w * cx * cz
    gyz = m2yz - nb * csy * csz + w * cy * cz
    cond_ref = _cond_from_gram(gxx, gyy, gzz, gxy, gxz, gyz)

    ux = ptsc_ref[:, 0:1]
    uy = ptsc_ref[:, 1:2]
    uz = ptsc_ref[:, 2:3]
    p1x = jnp.sum(sel * ux, axis=0, keepdims=True)
    p1y = jnp.sum(sel * uy, axis=0, keepdims=True)
    p1z = jnp.sum(sel * uz, axis=0, keepdims=True)
    p2xx = jnp.sum(sel * ux * ux, axis=0, keepdims=True)
    p2yy = jnp.sum(sel * uy * uy, axis=0, keepdims=True)
    p2zz = jnp.sum(sel * uz * uz, axis=0, keepdims=True)
    p2xy = jnp.sum(sel * ux * uy, axis=0, keepdims=True)
    p2xz = jnp.sum(sel * ux * uz, axis=0, keepdims=True)
    p2yz = jnp.sum(sel * uy * uz, axis=0, keepdims=True)
    cpx = p1x * nbinv
    cpy = p1y * nbinv
    cpz = p1z * nbinv
    hxx = p2xx - 2.0 * cpx * p1x + _NN * cpx * cpx
    hyy = p2yy - 2.0 * cpy * p1y + _NN * cpy * cpy
    hzz = p2zz - 2.0 * cpz * p1z + _NN * cpz * cpz
    hxy = p2xy - cpx * p1y - cpy * p1x + _NN * cpx * cpy
    hxz = p2xz - cpx * p1z - cpz * p1x + _NN * cpx * cpz
    hyz = p2yz - cpy * p1z - cpz * p1y + _NN * cpy * cpz
    cond_p = _cond_from_gram(hxx, hyy, hzz, hxy, hxz, hyz)

    diff = cond_p - cond_ref
    partial = jnp.sum(diff * diff, axis=1, keepdims=True)

    @pl.when(jnp.logical_and(b == 0, i == 0))
    def _():
        out_ref[0:1, 0:1] = jnp.zeros((1, 1), jnp.float32)

    out_ref[0:1, 0:1] = out_ref[0:1, 0:1] + partial

    @pl.when(jnp.logical_and(b == _B - 1, i == (_N // _RB) - 1))
    def _():
        out_ref[0:1, 0:1] = out_ref[0:1, 0:1] * (1.0 / (_B * _N))


def kernel(ref_points, points):
    ref_t = ref_points.transpose(0, 2, 1)
    out = pl.pallas_call(
        _body,
        grid=(_B, _N // _RB),
        in_specs=[
            pl.BlockSpec((None, _N, _C), lambda b, i: (b, 0, 0)),
            pl.BlockSpec((None, _C, _RB), lambda b, i: (b, 0, i)),
            pl.BlockSpec((None, _N, _C), lambda b, i: (b, 0, 0)),
        ],
        out_specs=pl.BlockSpec((1, 1), lambda b, i: (0, 0)),
        out_shape=jax.ShapeDtypeStruct((1, 1), jnp.float32),
    )(ref_points, ref_t, points)
    return out[0, 0]
